# baseline (speedup 1.0000x reference)
import functools

import jax
import jax.numpy as jnp
from jax import lax
from jax.experimental import pallas as pl
from jax.experimental.pallas import tpu as pltpu
from jax.experimental.pallas import tpu_sc as plsc

VOCAB = 1000000
D = 64
S = 512
B = 1024
N = B * S

NC = 2
NS = 16
NW = NC * NS
PER_W = N // NW
CHUNK = 512
N_CHUNKS = PER_W // CHUNK


@functools.partial(
    pl.kernel,
    mesh=plsc.VectorSubcoreMesh(core_axis_name="c", subcore_axis_name="s"),
    out_type=jax.ShapeDtypeStruct((N, D), jnp.float32),
    compiler_params=pltpu.CompilerParams(use_tc_tiling_on_sc=False),
    scratch_types=[
        pltpu.VMEM((CHUNK,), jnp.int32),
        pltpu.VMEM((CHUNK, D), jnp.float32),
        pltpu.VMEM((S, D), jnp.float32),
        pltpu.SemaphoreType.DMA,
    ],
)
def _sc_embed(x_hbm, tok_hbm, pos_hbm, out_hbm, idx_v, rows_v, pos_v, sem):
    wid = lax.axis_index("s") * NC + lax.axis_index("c")
    base = wid * PER_W

    pltpu.sync_copy(pos_hbm, pos_v)

    def chunk_body(c, _):
        off = base + c * CHUNK
        pltpu.sync_copy(x_hbm.at[pl.ds(off, CHUNK)], idx_v)
        pltpu.async_copy(tok_hbm.at[idx_v], rows_v, sem).wait()

        def add_body(r, _):
            for d in range(D // 16):
                sl = pl.ds(d * 16, 16)
                rows_v[r, sl] = rows_v[r, sl] + pos_v[r, sl]
            return ()

        lax.fori_loop(0, CHUNK, add_body, (), unroll=2)
        pltpu.sync_copy(rows_v, out_hbm.at[pl.ds(off, CHUNK)])
        return ()

    lax.fori_loop(0, N_CHUNKS, chunk_body, ())


def kernel(x, token_table, pos_table):
    xf = x.reshape(-1).astype(jnp.int32)
    out = _sc_embed(xf, token_table, pos_table)
    return out.reshape(B, S, D)


# speedup vs baseline: 1.0497x; 1.0497x over previous
import functools

import jax
import jax.numpy as jnp
from jax import lax
from jax.experimental import pallas as pl
from jax.experimental.pallas import tpu as pltpu
from jax.experimental.pallas import tpu_sc as plsc

VOCAB = 1000000
D = 64
S = 512
B = 1024
N = B * S

NC = 2
NS = 16
NW = NC * NS
PER_W = N // NW
CHUNK = 512
N_CHUNKS = PER_W // CHUNK


@functools.partial(
    pl.kernel,
    mesh=plsc.VectorSubcoreMesh(core_axis_name="c", subcore_axis_name="s"),
    out_type=jax.ShapeDtypeStruct((N, D), jnp.float32),
    compiler_params=pltpu.CompilerParams(use_tc_tiling_on_sc=False),
    scratch_types=[
        pltpu.VMEM((CHUNK,), jnp.int32),
        pltpu.VMEM((CHUNK,), jnp.int32),
        pltpu.VMEM((CHUNK, D), jnp.float32),
        pltpu.VMEM((CHUNK, D), jnp.float32),
        pltpu.VMEM((S, D), jnp.float32),
        pltpu.SemaphoreType.DMA,
        pltpu.SemaphoreType.DMA,
        pltpu.SemaphoreType.DMA,
        pltpu.SemaphoreType.DMA,
    ],
)
def _sc_embed(x_hbm, tok_hbm, pos_hbm, out_hbm, idx0, idx1, rows0, rows1,
              pos_v, g0, g1, s0, s1):
    wid = lax.axis_index("s") * NC + lax.axis_index("c")
    base = wid * PER_W
    idx_v = (idx0, idx1)
    rows_v = (rows0, rows1)
    gsem = (g0, g1)
    ssem = (s0, s1)

    pltpu.sync_copy(pos_hbm, pos_v)

    def fetch(g, b):
        off = base + g * CHUNK
        pltpu.sync_copy(x_hbm.at[pl.ds(off, CHUNK)], idx_v[b])
        pltpu.async_copy(tok_hbm.at[idx_v[b]], rows_v[b], gsem[b])

    fetch(0, 0)
    fetch(1, 1)

    def outer(c, _):
        for b in range(2):
            g = c * 2 + b
            rows = rows_v[b]
            pltpu.make_async_copy(tok_hbm.at[idx_v[b]], rows, gsem[b]).wait()

            def add_body(r, _):
                for d in range(D // 16):
                    sl = pl.ds(d * 16, 16)
                    rows[r, sl] = rows[r, sl] + pos_v[r, sl]
                return ()

            lax.fori_loop(0, CHUNK, add_body, (), unroll=4)

            off = base + g * CHUNK
            out_slice = out_hbm.at[pl.ds(off, CHUNK)]
            pltpu.async_copy(rows, out_slice, ssem[b])
            pltpu.make_async_copy(rows, out_slice, ssem[b]).wait()

            @pl.when(g + 2 < N_CHUNKS)
            def _():
                fetch(g + 2, b)
        return ()

    lax.fori_loop(0, N_CHUNKS // 2, outer, ())


def kernel(x, token_table, pos_table):
    xf = x.reshape(-1).astype(jnp.int32)
    out = _sc_embed(xf, token_table, pos_table)
    return out.reshape(B, S, D)
